# baseline (speedup 1.0000x reference)
import jax
import jax.numpy as jnp
from jax.experimental import pallas as pl
from jax.experimental.pallas import tpu as pltpu

_JORDER = (0, 2, 4, 6, 8, 10, 12, 14, 1, 3, 5, 7, 9, 11, 13)
_JINV = (0, 8, 1, 9, 2, 10, 3, 11, 4, 12, 5, 13, 6, 14, 7)
_JC = tuple(2 * j for j in _JORDER) + (0,) + \
      tuple(2 * j + 1 for j in _JORDER) + (1,)


def _pick_block(n, target):
    b = min(n, target)
    while n % b:
        b -= 1
    return b


def _c1_kernel(p_ref, w_ref, a1_ref):
    b = p_ref.shape[0]
    for i in range(15):
        pc = p_ref[:, i, :, :]
        z = jnp.dot(pc.reshape(b * 32, 54), w_ref[...],
                    preferred_element_type=jnp.float32)
        z = z.reshape(b, 32, 128)
        zh = jnp.maximum(z[:, :, 0:64], z[:, :, 64:128])
        zh = jnp.maximum(zh, 0.0)
        m = jnp.maximum(zh[:, 0:16, :], zh[:, 16:32, :])
        a1_ref[:, 16 * i:16 * i + 16, :] = m.astype(a1_ref.dtype)


def _im2col_pool1(x_nchw):
    n = x_nchw.shape[0]
    xh = jnp.transpose(x_nchw, (0, 2, 3, 1))
    cols = [xh[:, kh:kh + 30, kw:kw + 30, :]
            for kh in range(3) for kw in range(3)]
    x9 = jnp.concatenate(cols, axis=-1)
    imc = jnp.take(x9, jnp.asarray(_JC), axis=2)
    imc = imc.reshape(n, 15, 2, 32, 27)
    imc = jnp.swapaxes(imc, 2, 3).reshape(n, 15, 32, 54)
    return imc.astype(jnp.bfloat16)


def _c2_kernel(a_ref, w_ref, a2_ref):
    b = a_ref.shape[0]
    a0 = a_ref[...]
    t1 = jnp.concatenate([a0[:, :, 8:16], a0[:, :, 1:8], a0[:, :, 0:1]],
                         axis=2)
    t2 = jnp.concatenate([a0[:, :, 1:16], a0[:, :, 0:1]], axis=2)
    taps = (a0, t1, t2)
    for t in range(6):
        pieces = [taps[kw][:, kh + 2 * t:kh + 2 * t + 2, :, :]
                  for kh in range(3) for kw in range(3)]
        p = jnp.concatenate(pieces, axis=-1)
        z = jnp.dot(p.reshape(b * 32, 576), w_ref[...],
                    preferred_element_type=jnp.float32)
        z = jnp.maximum(z, 0.0).reshape(b, 2, 16, 128)
        zh = jnp.max(z, axis=1)
        a2_ref[:, t, :, :] = jnp.maximum(zh[:, 0:6, :], zh[:, 8:14, :])


def _c3_kernel(a_ref, wc_ref, w1_ref, b1_ref, w2_ref, b2_ref, w3_ref, b3_ref,
               exit_ref, logit_ref):
    b = a_ref.shape[0]
    a = a_ref[...]
    cols = [a[:, kh:kh + 4, kw:kw + 4, :]
            for kh in range(3) for kw in range(3)]
    p = jnp.concatenate(cols, axis=-1)
    z = jnp.dot(p.reshape(b * 16, 1152), wc_ref[...],
                preferred_element_type=jnp.float32)
    z = jnp.maximum(z, 0.0).reshape(b, 4, 4, 64)
    zh = jnp.max(z.reshape(b, 2, 2, 4, 64), axis=2)
    pooled = jnp.max(zh.reshape(b, 2, 2, 2, 64), axis=3)
    exit_ref[...] = pooled

    pr = pooled.reshape(b, 4, 64)
    h1 = jnp.dot(pr[:, 0, :], w1_ref[0], preferred_element_type=jnp.float32)
    for q in range(1, 4):
        h1 = h1 + jnp.dot(pr[:, q, :], w1_ref[q],
                          preferred_element_type=jnp.float32)
    h1 = h1 + b1_ref[...]
    h2 = jnp.dot(h1, w2_ref[...], preferred_element_type=jnp.float32) + b2_ref[...]
    h3 = jnp.dot(h2, w3_ref[...], preferred_element_type=jnp.float32) + b3_ref[...]
    logit_ref[...] = h3


def kernel(x_nchw, w_conv1, w_conv2, w_conv3,
           w_fc1, b_fc1, w_fc2, b_fc2, w_fc3, b_fc3):
    N = x_nchw.shape[0]
    imcol1 = _im2col_pool1(x_nchw)

    w1m = w_conv1.reshape(27, 64).astype(jnp.bfloat16)
    zpad = jnp.zeros((27, 64), jnp.bfloat16)
    wbd = jnp.concatenate([
        jnp.concatenate([w1m, zpad], axis=1),
        jnp.concatenate([zpad, w1m], axis=1)], axis=0)
    w2m = w_conv2.reshape(576, 128).astype(jnp.bfloat16)
    w3m = w_conv3.reshape(1152, 64)
    H1, H2, OUT = w_fc1.shape[1], w_fc2.shape[1], w_fc3.shape[1]
    OUT_PAD = 128
    w1r = jnp.transpose(w_fc1.reshape(64, 4, H1), (1, 0, 2))
    w3p = jnp.pad(w_fc3, ((0, 0), (0, OUT_PAD - OUT)))
    b3p = jnp.pad(b_fc3, ((0, 0), (0, OUT_PAD - OUT)))

    B1 = _pick_block(N, 64)
    a1 = pl.pallas_call(
        _c1_kernel,
        out_shape=jax.ShapeDtypeStruct((N, 240, 64), jnp.bfloat16),
        grid=(N // B1,),
        in_specs=[
            pl.BlockSpec((B1, 15, 32, 54), lambda i: (i, 0, 0, 0)),
            pl.BlockSpec((54, 128), lambda i: (0, 0)),
        ],
        out_specs=pl.BlockSpec((B1, 240, 64), lambda i: (i, 0, 0)),
        compiler_params=pltpu.CompilerParams(
            dimension_semantics=("parallel",)),
    )(imcol1, wbd)

    a1pg = a1.reshape(N, 15, 16, 64)

    B2 = _pick_block(N, 64)
    a2 = pl.pallas_call(
        _c2_kernel,
        out_shape=jax.ShapeDtypeStruct((N, 6, 6, 128), jnp.float32),
        grid=(N // B2,),
        in_specs=[
            pl.BlockSpec((B2, 15, 16, 64), lambda i: (i, 0, 0, 0)),
            pl.BlockSpec((576, 128), lambda i: (0, 0)),
        ],
        out_specs=pl.BlockSpec((B2, 6, 6, 128), lambda i: (i, 0, 0, 0)),
        compiler_params=pltpu.CompilerParams(
            dimension_semantics=("parallel",)),
    )(a1pg, w2m)

    B3 = _pick_block(N, 128)
    a3, logits = pl.pallas_call(
        _c3_kernel,
        out_shape=(
            jax.ShapeDtypeStruct((N, 2, 2, 64), jnp.float32),
            jax.ShapeDtypeStruct((N, OUT_PAD), jnp.float32),
        ),
        grid=(N // B3,),
        in_specs=[
            pl.BlockSpec((B3, 6, 6, 128), lambda i: (i, 0, 0, 0)),
            pl.BlockSpec((1152, 64), lambda i: (0, 0)),
            pl.BlockSpec((4, 64, H1), lambda i: (0, 0, 0)),
            pl.BlockSpec((1, H1), lambda i: (0, 0)),
            pl.BlockSpec((H1, H2), lambda i: (0, 0)),
            pl.BlockSpec((1, H2), lambda i: (0, 0)),
            pl.BlockSpec((H2, OUT_PAD), lambda i: (0, 0)),
            pl.BlockSpec((1, OUT_PAD), lambda i: (0, 0)),
        ],
        out_specs=(
            pl.BlockSpec((B3, 2, 2, 64), lambda i: (i, 0, 0, 0)),
            pl.BlockSpec((B3, OUT_PAD), lambda i: (i, 0)),
        ),
        compiler_params=pltpu.CompilerParams(
            dimension_semantics=("parallel",)),
    )(a2, w3m, w1r, b_fc1, w_fc2, b_fc2, w3p, b3p)

    idx = [(16 * i + _JINV[j]) * 64 + c
           for c in range(64) for i in range(15) for j in range(15)]
    e1 = jnp.take(a1.reshape(N, 240 * 64), jnp.asarray(idx), axis=1)
    e1 = e1.reshape(N, 64, 15, 15).astype(jnp.float32)
    e2 = jnp.transpose(a2, (0, 3, 1, 2))
    e3 = jnp.transpose(a3, (0, 3, 1, 2))
    return logits[:, :OUT], [e1, e2, e3]


# speedup vs baseline: 1.0614x; 1.0614x over previous
import jax
import jax.numpy as jnp
from jax.experimental import pallas as pl
from jax.experimental.pallas import tpu as pltpu

_JORDER = (0, 2, 4, 6, 8, 10, 12, 14, 1, 3, 5, 7, 9, 11, 13)
_JINV = (0, 8, 1, 9, 2, 10, 3, 11, 4, 12, 5, 13, 6, 14, 7)
_JC = tuple(2 * j for j in _JORDER) + (0,) + \
      tuple(2 * j + 1 for j in _JORDER) + (1,)


def _pick_block(n, target):
    b = min(n, target)
    while n % b:
        b -= 1
    return b


def _c1_kernel(p_ref, w_ref, a1_ref):
    b = p_ref.shape[0]
    for i in range(15):
        pc = p_ref[:, i, :, :]
        z = jnp.dot(pc.reshape(b * 32, 54), w_ref[...],
                    preferred_element_type=jnp.float32)
        z = z.reshape(b, 32, 128)
        zh = jnp.maximum(z[:, :, 0:64], z[:, :, 64:128])
        zh = jnp.maximum(zh, 0.0)
        m = jnp.maximum(zh[:, 0:16, :], zh[:, 16:32, :])
        a1_ref[:, 16 * i:16 * i + 16, :] = m.astype(a1_ref.dtype)


def _im2col_pool1(x_nchw):
    n = x_nchw.shape[0]
    xh = jnp.transpose(x_nchw, (0, 2, 3, 1))
    cols = [xh[:, kh:kh + 30, kw:kw + 30, :]
            for kh in range(3) for kw in range(3)]
    x9 = jnp.concatenate(cols, axis=-1)
    imc = jnp.take(x9, jnp.asarray(_JC), axis=2)
    imc = imc.reshape(n, 15, 2, 32, 27)
    imc = jnp.swapaxes(imc, 2, 3).reshape(n, 15, 32, 54)
    return imc.astype(jnp.bfloat16)


def _c2_kernel(a_ref, w_ref, a2_ref):
    b = a_ref.shape[0]
    a0 = a_ref[...]
    t1 = jnp.concatenate([a0[:, :, 8:16], a0[:, :, 1:8], a0[:, :, 0:1]],
                         axis=2)
    t2 = jnp.concatenate([a0[:, :, 1:16], a0[:, :, 0:1]], axis=2)
    taps = (a0, t1, t2)
    for t in range(6):
        pieces = [taps[kw][:, kh + 2 * t:kh + 2 * t + 2, :, :]
                  for kh in range(3) for kw in range(3)]
        p = jnp.concatenate(pieces, axis=-1)
        z = jnp.dot(p.reshape(b * 32, 576), w_ref[...],
                    preferred_element_type=jnp.float32)
        z = jnp.maximum(z, 0.0).reshape(b, 2, 16, 128)
        zh = jnp.max(z, axis=1)
        a2_ref[:, t, :, :] = jnp.maximum(zh[:, 0:6, :], zh[:, 8:14, :])


def _c3_kernel(a_ref, wc_ref, w1_ref, b1_ref, w2_ref, b2_ref, w3_ref, b3_ref,
               exit_ref, logit_ref):
    b = a_ref.shape[0]
    a = a_ref[...]
    cols = [a[:, kh:kh + 4, kw:kw + 4, :]
            for kh in range(3) for kw in range(3)]
    p = jnp.concatenate(cols, axis=-1)
    z = jnp.dot(p.reshape(b * 16, 1152), wc_ref[...],
                preferred_element_type=jnp.float32)
    z = jnp.maximum(z, 0.0).reshape(b, 4, 4, 64)
    zh = jnp.max(z.reshape(b, 2, 2, 4, 64), axis=2)
    pooled = jnp.max(zh.reshape(b, 2, 2, 2, 64), axis=3)
    exit_ref[...] = pooled

    pr = pooled.reshape(b, 4, 64)
    h1 = jnp.dot(pr[:, 0, :], w1_ref[0], preferred_element_type=jnp.float32)
    for q in range(1, 4):
        h1 = h1 + jnp.dot(pr[:, q, :], w1_ref[q],
                          preferred_element_type=jnp.float32)
    h1 = h1 + b1_ref[...]
    h2 = jnp.dot(h1, w2_ref[...], preferred_element_type=jnp.float32) + b2_ref[...]
    h3 = jnp.dot(h2, w3_ref[...], preferred_element_type=jnp.float32) + b3_ref[...]
    logit_ref[...] = h3


def kernel(x_nchw, w_conv1, w_conv2, w_conv3,
           w_fc1, b_fc1, w_fc2, b_fc2, w_fc3, b_fc3):
    N = x_nchw.shape[0]
    imcol1 = _im2col_pool1(x_nchw)

    w1m = w_conv1.reshape(27, 64).astype(jnp.bfloat16)
    zpad = jnp.zeros((27, 64), jnp.bfloat16)
    wbd = jnp.concatenate([
        jnp.concatenate([w1m, zpad], axis=1),
        jnp.concatenate([zpad, w1m], axis=1)], axis=0)
    w2m = w_conv2.reshape(576, 128).astype(jnp.bfloat16)
    w3m = w_conv3.reshape(1152, 64)
    H1, H2, OUT = w_fc1.shape[1], w_fc2.shape[1], w_fc3.shape[1]
    OUT_PAD = 128
    w1r = jnp.transpose(w_fc1.reshape(64, 4, H1), (1, 0, 2))
    w3p = jnp.pad(w_fc3, ((0, 0), (0, OUT_PAD - OUT)))
    b3p = jnp.pad(b_fc3, ((0, 0), (0, OUT_PAD - OUT)))

    B1 = _pick_block(N, 64)
    a1 = pl.pallas_call(
        _c1_kernel,
        out_shape=jax.ShapeDtypeStruct((N, 240, 64), jnp.bfloat16),
        grid=(N // B1,),
        in_specs=[
            pl.BlockSpec((B1, 15, 32, 54), lambda i: (i, 0, 0, 0)),
            pl.BlockSpec((54, 128), lambda i: (0, 0)),
        ],
        out_specs=pl.BlockSpec((B1, 240, 64), lambda i: (i, 0, 0)),
        compiler_params=pltpu.CompilerParams(
            dimension_semantics=("parallel",)),
    )(imcol1, wbd)

    a1pg = a1.reshape(N, 15, 16, 64)

    B2 = _pick_block(N, 64)
    a2 = pl.pallas_call(
        _c2_kernel,
        out_shape=jax.ShapeDtypeStruct((N, 6, 6, 128), jnp.float32),
        grid=(N // B2,),
        in_specs=[
            pl.BlockSpec((B2, 15, 16, 64), lambda i: (i, 0, 0, 0)),
            pl.BlockSpec((576, 128), lambda i: (0, 0)),
        ],
        out_specs=pl.BlockSpec((B2, 6, 6, 128), lambda i: (i, 0, 0, 0)),
        compiler_params=pltpu.CompilerParams(
            dimension_semantics=("parallel",)),
    )(a1pg, w2m)

    B3 = _pick_block(N, 128)
    a3, logits = pl.pallas_call(
        _c3_kernel,
        out_shape=(
            jax.ShapeDtypeStruct((N, 2, 2, 64), jnp.float32),
            jax.ShapeDtypeStruct((N, OUT_PAD), jnp.float32),
        ),
        grid=(N // B3,),
        in_specs=[
            pl.BlockSpec((B3, 6, 6, 128), lambda i: (i, 0, 0, 0)),
            pl.BlockSpec((1152, 64), lambda i: (0, 0)),
            pl.BlockSpec((4, 64, H1), lambda i: (0, 0, 0)),
            pl.BlockSpec((1, H1), lambda i: (0, 0)),
            pl.BlockSpec((H1, H2), lambda i: (0, 0)),
            pl.BlockSpec((1, H2), lambda i: (0, 0)),
            pl.BlockSpec((H2, OUT_PAD), lambda i: (0, 0)),
            pl.BlockSpec((1, OUT_PAD), lambda i: (0, 0)),
        ],
        out_specs=(
            pl.BlockSpec((B3, 2, 2, 64), lambda i: (i, 0, 0, 0)),
            pl.BlockSpec((B3, OUT_PAD), lambda i: (i, 0)),
        ),
        compiler_params=pltpu.CompilerParams(
            dimension_semantics=("parallel",)),
    )(a2, w3m, w1r, b_fc1, w_fc2, b_fc2, w3p, b3p)

    a1n = jnp.take(a1pg, jnp.asarray(_JINV), axis=2)
    e1 = jnp.transpose(a1n, (0, 3, 1, 2)).astype(jnp.float32)
    e2 = jnp.transpose(a2, (0, 3, 1, 2))
    e3 = jnp.transpose(a3, (0, 3, 1, 2))
    return logits[:, :OUT], [e1, e2, e3]
